# baseline (speedup 1.0000x reference)
import functools

import jax
import jax.numpy as jnp
from jax import lax
from jax.experimental import pallas as pl
from jax.experimental.pallas import tpu as pltpu
from jax.experimental.pallas import tpu_sc as plsc

N = 262144
NB = 100
H = 0.1
EPS = 1e-10
NC = 2
NS = 16
L = 16
NW = NC * NS
SHARD = N // NS
HALF = SHARD // NC
ROW = 320
COFF = 104
NOUT = 104
CUT = 0.6
UNROLL = 8


def _sc_hist(q, p):
    mesh = plsc.VectorSubcoreMesh(core_axis_name="c", subcore_axis_name="s")

    @functools.partial(
        pl.kernel,
        out_type=[
            jax.ShapeDtypeStruct((NW * NOUT,), jnp.float32),
            jax.ShapeDtypeStruct((NW * NOUT,), jnp.float32),
        ],
        mesh=mesh,
        compiler_params=pltpu.CompilerParams(needs_layout_passes=False),
        scratch_types=[
            pltpu.VMEM((SHARD,), jnp.float32),
            pltpu.VMEM((SHARD,), jnp.float32),
            pltpu.VMEM((L * ROW,), jnp.float32),
            pltpu.VMEM((L * ROW,), jnp.float32),
            pltpu.VMEM((2 * L,), jnp.float32),
            pltpu.VMEM((NS * 2 * L,), jnp.float32),
            pltpu.VMEM((ROW,), jnp.float32),
            pltpu.VMEM_SHARED((NS * 2 * L,), jnp.float32),
        ],
    )
    def hist_kernel(q_hbm, p_hbm, oq_hbm, op_hbm, xq_v, xp_v, hq_v, hp_v,
                    mm_v, allmm_v, csum_v, shared_mm):
        c = lax.axis_index("c")
        s = lax.axis_index("s")

        pltpu.sync_copy(q_hbm.at[pl.ds(s * SHARD, SHARD)], xq_v)
        pltpu.sync_copy(p_hbm.at[pl.ds(s * SHARD, SHARD)], xp_v)

        def mm_body(i, carry):
            mn, mx = carry
            for j in range(4):
                a = xq_v[pl.ds((i * 4 + j) * L, L)]
                b = xp_v[pl.ds((i * 4 + j) * L, L)]
                mn = jnp.minimum(mn, jnp.minimum(a, b))
                mx = jnp.maximum(mx, jnp.maximum(a, b))
            return (mn, mx)

        first = xq_v[pl.ds(0, L)]
        mn, mx = lax.fori_loop(0, SHARD // L // 4, mm_body, (first, first))
        mm_v[pl.ds(0, L)] = mn
        mm_v[pl.ds(L, L)] = mx

        pltpu.sync_copy(mm_v, shared_mm.at[pl.ds(s * 2 * L, 2 * L)])
        plsc.subcore_barrier()
        pltpu.sync_copy(shared_mm, allmm_v)
        amn = allmm_v[pl.ds(0, L)]
        amx = allmm_v[pl.ds(L, L)]
        for t in range(1, NS):
            amn = jnp.minimum(amn, allmm_v[pl.ds(t * 2 * L, L)])
            amx = jnp.maximum(amx, allmm_v[pl.ds(t * 2 * L + L, L)])
        lane = lax.iota(jnp.int32, L)
        for sh in (8, 4, 2, 1):
            perm = (lane + sh) & (L - 1)
            amn = jnp.minimum(amn, amn.at[perm].get(mode="promise_in_bounds"))
            amx = jnp.maximum(amx, amx.at[perm].get(mode="promise_in_bounds"))
        gmn = amn
        rng = amx - amn
        delta = rng * (1.0 / (NB - 1))
        invd = (NB - 1.0) / rng
        dh = delta * (1.0 / H)
        rad = jnp.minimum(CUT / delta, 200.0)
        ri_v = jnp.minimum(rad.astype(jnp.int32) + 1, NB - 1)
        rf = ri_v.astype(jnp.float32)
        ri = ri_v[0]
        ngroups = (2 * ri + UNROLL) >> 3

        zeros = jnp.zeros((L,), jnp.float32)

        def z_body(i, _):
            hq_v[pl.ds(i * L, L)] = zeros
            hp_v[pl.ds(i * L, L)] = zeros
            return 0

        lax.fori_loop(0, ROW, z_body, 0)

        laneoff = lax.iota(jnp.int32, L) * ROW
        base = c * HALF

        CH = 2

        def chunk(i, _):
            st = []
            for j in range(CH):
                off = base + (i * CH + j) * L
                for x_v in (xq_v, xp_v):
                    x = x_v[pl.ds(off, L)]
                    u = (x - gmn) * invd
                    k0 = (u + 0.5).astype(jnp.int32)
                    z = (u - k0.astype(jnp.float32) + rf) * dh
                    idx = laneoff + (k0 + (COFF - ri_v))
                    st += [z, idx]

            def wgroup(g, carry):
                cs = list(carry)
                for _ in range(UNROLL):
                    for m in range(2 * CH):
                        h_v = hq_v if m % 2 == 0 else hp_v
                        v = jnp.exp(cs[2 * m] * cs[2 * m] * (-0.5))
                        plsc.addupdate_scatter(h_v, [cs[2 * m + 1]], v)
                        cs[2 * m] = cs[2 * m] - dh
                        cs[2 * m + 1] = cs[2 * m + 1] + 1
                return tuple(cs)

            lax.fori_loop(0, ngroups, wgroup, tuple(st))
            return 0

        lax.fori_loop(0, HALF // L // CH, chunk, 0)

        wid = c * NS + s

        def write_out(h_v, o_hbm):
            for cc in range(6, 13):
                acc = h_v[pl.ds(cc * L, L)]
                for lane in range(1, L):
                    acc = acc + h_v[pl.ds(lane * ROW + cc * L, L)]
                csum_v[pl.ds(cc * L, L)] = acc
            pltpu.sync_copy(csum_v.at[pl.ds(COFF, NOUT)], o_hbm.at[pl.ds(wid * NOUT, NOUT)])

        write_out(hq_v, oq_hbm)
        write_out(hp_v, op_hbm)

    return hist_kernel(q, p)


def _tc_jsd(oq, op):
    def body(hq_ref, hp_ref, o_ref):
        sq = jnp.sum(hq_ref[...], axis=0, keepdims=True)
        sp = jnp.sum(hp_ref[...], axis=0, keepdims=True)
        colid = lax.broadcasted_iota(jnp.int32, (1, NOUT), 1)
        mask = colid < NB
        pdfq = jnp.where(mask, sq * (1.0 / N), 0.0)
        pdfp = jnp.where(mask, sp * (1.0 / N), 0.0)
        qh = pdfq / (jnp.sum(pdfq) + EPS)
        ph = pdfp / (jnp.sum(pdfp) + EPS)
        m = 0.5 * (ph + qh)
        qh = jnp.maximum(qh, 1e-45)
        ph = jnp.maximum(ph, 1e-45)
        m = jnp.maximum(m, 1e-45)
        lp = jnp.log(ph)
        lq = jnp.log(qh)
        lm = jnp.log(m)
        t = jnp.exp(lp) * (lp - lm) + jnp.exp(lq) * (lq - lm)
        o_ref[...] = 0.5 * jnp.sum(jnp.where(mask, t, 0.0), keepdims=True)

    return pl.pallas_call(
        body,
        out_shape=jax.ShapeDtypeStruct((1, 1), jnp.float32),
    )(oq, op)


def kernel(q, p):
    oq, op = _sc_hist(q, p)
    return _tc_jsd(oq.reshape(NW, NOUT), op.reshape(NW, NOUT))[0, 0]


# speedup vs baseline: 1.1937x; 1.1937x over previous
import functools

import jax
import jax.numpy as jnp
from jax import lax
from jax.experimental import pallas as pl
from jax.experimental.pallas import tpu as pltpu
from jax.experimental.pallas import tpu_sc as plsc

N = 262144
NB = 100
H = 0.1
EPS = 1e-10
NC = 2
NS = 16
L = 16
NW = NC * NS
SHARD = N // NS
HALF = SHARD // NC
ROW = 320
COFF = 104
NOUT = 104
CUT = 0.6
UNROLL = 8


def _sc_hist(q, p):
    mesh = plsc.VectorSubcoreMesh(core_axis_name="c", subcore_axis_name="s")

    @functools.partial(
        pl.kernel,
        out_type=[
            jax.ShapeDtypeStruct((NW * NOUT,), jnp.float32),
            jax.ShapeDtypeStruct((NW * NOUT,), jnp.float32),
        ],
        mesh=mesh,
        compiler_params=pltpu.CompilerParams(needs_layout_passes=False),
        scratch_types=[
            pltpu.VMEM((SHARD,), jnp.float32),
            pltpu.VMEM((SHARD,), jnp.float32),
            pltpu.VMEM((L * ROW,), jnp.float32),
            pltpu.VMEM((L * ROW,), jnp.float32),
            pltpu.VMEM((2 * L,), jnp.float32),
            pltpu.VMEM((NS * 2 * L,), jnp.float32),
            pltpu.VMEM((ROW,), jnp.float32),
            pltpu.VMEM_SHARED((NS * 2 * L,), jnp.float32),
        ],
    )
    def hist_kernel(q_hbm, p_hbm, oq_hbm, op_hbm, xq_v, xp_v, hq_v, hp_v,
                    mm_v, allmm_v, csum_v, shared_mm):
        c = lax.axis_index("c")
        s = lax.axis_index("s")

        pltpu.sync_copy(q_hbm.at[pl.ds(s * SHARD, SHARD)], xq_v)
        pltpu.sync_copy(p_hbm.at[pl.ds(s * SHARD, SHARD)], xp_v)

        def mm_body(i, carry):
            mn, mx = carry
            for j in range(4):
                a = xq_v[pl.ds((i * 4 + j) * L, L)]
                b = xp_v[pl.ds((i * 4 + j) * L, L)]
                mn = jnp.minimum(mn, jnp.minimum(a, b))
                mx = jnp.maximum(mx, jnp.maximum(a, b))
            return (mn, mx)

        first = xq_v[pl.ds(0, L)]
        mn, mx = lax.fori_loop(0, SHARD // L // 4, mm_body, (first, first))
        mm_v[pl.ds(0, L)] = mn
        mm_v[pl.ds(L, L)] = mx

        pltpu.sync_copy(mm_v, shared_mm.at[pl.ds(s * 2 * L, 2 * L)])
        plsc.subcore_barrier()
        pltpu.sync_copy(shared_mm, allmm_v)
        amn = allmm_v[pl.ds(0, L)]
        amx = allmm_v[pl.ds(L, L)]
        for t in range(1, NS):
            amn = jnp.minimum(amn, allmm_v[pl.ds(t * 2 * L, L)])
            amx = jnp.maximum(amx, allmm_v[pl.ds(t * 2 * L + L, L)])
        lane = lax.iota(jnp.int32, L)
        for sh in (8, 4, 2, 1):
            perm = (lane + sh) & (L - 1)
            amn = jnp.minimum(amn, amn.at[perm].get(mode="promise_in_bounds"))
            amx = jnp.maximum(amx, amx.at[perm].get(mode="promise_in_bounds"))
        gmn = amn
        rng = amx - amn
        delta = rng * (1.0 / (NB - 1))
        invd = (NB - 1.0) / rng
        dh = delta * (1.0 / H)
        rad = jnp.minimum(CUT / delta, 200.0)
        ri_v = jnp.minimum(rad.astype(jnp.int32) + 1, NB - 1)
        rf = ri_v.astype(jnp.float32)
        ri = ri_v[0]
        ngroups = (2 * ri + UNROLL) >> 3

        zeros = jnp.zeros((L,), jnp.float32)

        def z_body(i, _):
            hq_v[pl.ds(i * L, L)] = zeros
            hp_v[pl.ds(i * L, L)] = zeros
            return 0

        lax.fori_loop(0, ROW, z_body, 0)

        laneoff = lax.iota(jnp.int32, L) * ROW
        base = c * HALF

        CH = 2
        cmul = jnp.exp(-dh * dh)

        def chunk(i, _):
            st = []
            for j in range(CH):
                off = base + (i * CH + j) * L
                for x_v in (xq_v, xp_v):
                    x = x_v[pl.ds(off, L)]
                    u = (x - gmn) * invd
                    k0 = (u + 0.5).astype(jnp.int32)
                    z = (u - k0.astype(jnp.float32) + rf) * dh
                    idx = laneoff + (k0 + (COFF - ri_v))
                    f = jnp.exp(z * z * (-0.5))
                    r = jnp.exp(jnp.minimum((z - 0.5 * dh) * dh, 60.0))
                    st += [f, r, idx]

            def wgroup(g, carry):
                cs = list(carry)
                for _ in range(UNROLL):
                    for m in range(2 * CH):
                        h_v = hq_v if m % 2 == 0 else hp_v
                        plsc.addupdate_scatter(h_v, [cs[3 * m + 2]], cs[3 * m])
                        cs[3 * m] = cs[3 * m] * cs[3 * m + 1]
                        cs[3 * m + 1] = cs[3 * m + 1] * cmul
                        cs[3 * m + 2] = cs[3 * m + 2] + 1
                return tuple(cs)

            lax.fori_loop(0, ngroups, wgroup, tuple(st))
            return 0

        lax.fori_loop(0, HALF // L // CH, chunk, 0)

        wid = c * NS + s

        def write_out(h_v, o_hbm):
            for cc in range(6, 13):
                acc = h_v[pl.ds(cc * L, L)]
                for lane in range(1, L):
                    acc = acc + h_v[pl.ds(lane * ROW + cc * L, L)]
                csum_v[pl.ds(cc * L, L)] = acc
            pltpu.sync_copy(csum_v.at[pl.ds(COFF, NOUT)], o_hbm.at[pl.ds(wid * NOUT, NOUT)])

        write_out(hq_v, oq_hbm)
        write_out(hp_v, op_hbm)

    return hist_kernel(q, p)


def _tc_jsd(oq, op):
    def body(hq_ref, hp_ref, o_ref):
        sq = jnp.sum(hq_ref[...], axis=0, keepdims=True)
        sp = jnp.sum(hp_ref[...], axis=0, keepdims=True)
        colid = lax.broadcasted_iota(jnp.int32, (1, NOUT), 1)
        mask = colid < NB
        pdfq = jnp.where(mask, sq * (1.0 / N), 0.0)
        pdfp = jnp.where(mask, sp * (1.0 / N), 0.0)
        qh = pdfq / (jnp.sum(pdfq) + EPS)
        ph = pdfp / (jnp.sum(pdfp) + EPS)
        m = 0.5 * (ph + qh)
        qh = jnp.maximum(qh, 1e-45)
        ph = jnp.maximum(ph, 1e-45)
        m = jnp.maximum(m, 1e-45)
        lp = jnp.log(ph)
        lq = jnp.log(qh)
        lm = jnp.log(m)
        t = jnp.exp(lp) * (lp - lm) + jnp.exp(lq) * (lq - lm)
        o_ref[...] = 0.5 * jnp.sum(jnp.where(mask, t, 0.0), keepdims=True)

    return pl.pallas_call(
        body,
        out_shape=jax.ShapeDtypeStruct((1, 1), jnp.float32),
    )(oq, op)


def kernel(q, p):
    oq, op = _sc_hist(q, p)
    return _tc_jsd(oq.reshape(NW, NOUT), op.reshape(NW, NOUT))[0, 0]


# speedup vs baseline: 1.2508x; 1.0478x over previous
import functools

import jax
import jax.numpy as jnp
from jax import lax
from jax.experimental import pallas as pl
from jax.experimental.pallas import tpu as pltpu
from jax.experimental.pallas import tpu_sc as plsc

N = 262144
NB = 100
H = 0.1
EPS = 1e-10
NC = 2
NS = 16
L = 16
NW = NC * NS
SHARD = N // NS
HALF = SHARD // NC
ROW = 320
COFF = 104
NOUT = 104
CUT = 0.6
UNROLL = 16


def _sc_hist(q, p):
    mesh = plsc.VectorSubcoreMesh(core_axis_name="c", subcore_axis_name="s")

    @functools.partial(
        pl.kernel,
        out_type=[
            jax.ShapeDtypeStruct((NW * NOUT,), jnp.float32),
            jax.ShapeDtypeStruct((NW * NOUT,), jnp.float32),
        ],
        mesh=mesh,
        compiler_params=pltpu.CompilerParams(needs_layout_passes=False),
        scratch_types=[
            pltpu.VMEM((SHARD,), jnp.float32),
            pltpu.VMEM((SHARD,), jnp.float32),
            pltpu.VMEM((L * ROW,), jnp.float32),
            pltpu.VMEM((L * ROW,), jnp.float32),
            pltpu.VMEM((2 * L,), jnp.float32),
            pltpu.VMEM((NS * 2 * L,), jnp.float32),
            pltpu.VMEM((ROW,), jnp.float32),
            pltpu.VMEM_SHARED((NS * 2 * L,), jnp.float32),
        ],
    )
    def hist_kernel(q_hbm, p_hbm, oq_hbm, op_hbm, xq_v, xp_v, hq_v, hp_v,
                    mm_v, allmm_v, csum_v, shared_mm):
        c = lax.axis_index("c")
        s = lax.axis_index("s")

        pltpu.sync_copy(q_hbm.at[pl.ds(s * SHARD, SHARD)], xq_v)
        pltpu.sync_copy(p_hbm.at[pl.ds(s * SHARD, SHARD)], xp_v)

        def mm_body(i, carry):
            mn, mx = carry
            for j in range(4):
                a = xq_v[pl.ds((i * 4 + j) * L, L)]
                b = xp_v[pl.ds((i * 4 + j) * L, L)]
                mn = jnp.minimum(mn, jnp.minimum(a, b))
                mx = jnp.maximum(mx, jnp.maximum(a, b))
            return (mn, mx)

        first = xq_v[pl.ds(0, L)]
        mn, mx = lax.fori_loop(0, SHARD // L // 4, mm_body, (first, first))
        mm_v[pl.ds(0, L)] = mn
        mm_v[pl.ds(L, L)] = mx

        pltpu.sync_copy(mm_v, shared_mm.at[pl.ds(s * 2 * L, 2 * L)])
        plsc.subcore_barrier()
        pltpu.sync_copy(shared_mm, allmm_v)
        amn = allmm_v[pl.ds(0, L)]
        amx = allmm_v[pl.ds(L, L)]
        for t in range(1, NS):
            amn = jnp.minimum(amn, allmm_v[pl.ds(t * 2 * L, L)])
            amx = jnp.maximum(amx, allmm_v[pl.ds(t * 2 * L + L, L)])
        lane = lax.iota(jnp.int32, L)
        for sh in (8, 4, 2, 1):
            perm = (lane + sh) & (L - 1)
            amn = jnp.minimum(amn, amn.at[perm].get(mode="promise_in_bounds"))
            amx = jnp.maximum(amx, amx.at[perm].get(mode="promise_in_bounds"))
        gmn = amn
        rng = amx - amn
        delta = rng * (1.0 / (NB - 1))
        invd = (NB - 1.0) / rng
        dh = delta * (1.0 / H)
        rad = jnp.minimum(CUT / delta, 200.0)
        ri_v = jnp.minimum(rad.astype(jnp.int32) + 1, NB - 1)
        rf = ri_v.astype(jnp.float32)
        ri = ri_v[0]
        ngroups = (2 * ri + UNROLL) >> 4

        zeros = jnp.zeros((L,), jnp.float32)

        def z_body(i, _):
            hq_v[pl.ds(i * L, L)] = zeros
            hp_v[pl.ds(i * L, L)] = zeros
            return 0

        lax.fori_loop(0, ROW, z_body, 0)

        laneoff = lax.iota(jnp.int32, L) * ROW
        base = c * HALF

        CH = 4
        cmul = jnp.exp(-dh * dh)
        hdh2 = 0.5 * (dh * dh)
        base_off = laneoff + (COFF - ri_v)

        def chunk(i, _):
            st = []
            for j in range(CH):
                off = base + (i * CH + j) * L
                for x_v in (xq_v, xp_v):
                    x = x_v[pl.ds(off, L)]
                    u = (x - gmn) * invd
                    k0 = (u + 0.5).astype(jnp.int32)
                    z = (u - k0.astype(jnp.float32) + rf) * dh
                    idx = base_off + k0
                    f = jnp.exp(z * z * (-0.5))
                    r = jnp.exp(jnp.minimum(z * dh - hdh2, 60.0))
                    st += [f, r, idx]

            def wgroup(g, carry):
                cs = list(carry)
                for _ in range(UNROLL):
                    for m in range(2 * CH):
                        h_v = hq_v if m % 2 == 0 else hp_v
                        plsc.addupdate_scatter(h_v, [cs[3 * m + 2]], cs[3 * m])
                        cs[3 * m] = cs[3 * m] * cs[3 * m + 1]
                        cs[3 * m + 1] = cs[3 * m + 1] * cmul
                        cs[3 * m + 2] = cs[3 * m + 2] + 1
                return tuple(cs)

            lax.fori_loop(0, ngroups, wgroup, tuple(st))
            return 0

        lax.fori_loop(0, HALF // L // CH, chunk, 0)

        wid = c * NS + s

        def write_out(h_v, o_hbm):
            for cc in range(6, 13):
                acc = h_v[pl.ds(cc * L, L)]
                for lane in range(1, L):
                    acc = acc + h_v[pl.ds(lane * ROW + cc * L, L)]
                csum_v[pl.ds(cc * L, L)] = acc
            pltpu.sync_copy(csum_v.at[pl.ds(COFF, NOUT)], o_hbm.at[pl.ds(wid * NOUT, NOUT)])

        write_out(hq_v, oq_hbm)
        write_out(hp_v, op_hbm)

    return hist_kernel(q, p)


def _tc_jsd(oq, op):
    def body(hq_ref, hp_ref, o_ref):
        sq = jnp.sum(hq_ref[...], axis=0, keepdims=True)
        sp = jnp.sum(hp_ref[...], axis=0, keepdims=True)
        colid = lax.broadcasted_iota(jnp.int32, (1, NOUT), 1)
        mask = colid < NB
        pdfq = jnp.where(mask, sq * (1.0 / N), 0.0)
        pdfp = jnp.where(mask, sp * (1.0 / N), 0.0)
        qh = pdfq / (jnp.sum(pdfq) + EPS)
        ph = pdfp / (jnp.sum(pdfp) + EPS)
        m = 0.5 * (ph + qh)
        qh = jnp.maximum(qh, 1e-45)
        ph = jnp.maximum(ph, 1e-45)
        m = jnp.maximum(m, 1e-45)
        lp = jnp.log(ph)
        lq = jnp.log(qh)
        lm = jnp.log(m)
        t = jnp.exp(lp) * (lp - lm) + jnp.exp(lq) * (lq - lm)
        o_ref[...] = 0.5 * jnp.sum(jnp.where(mask, t, 0.0), keepdims=True)

    return pl.pallas_call(
        body,
        out_shape=jax.ShapeDtypeStruct((1, 1), jnp.float32),
    )(oq, op)


def kernel(q, p):
    oq, op = _sc_hist(q, p)
    return _tc_jsd(oq.reshape(NW, NOUT), op.reshape(NW, NOUT))[0, 0]
